# baseline (speedup 1.0000x reference)
import functools

import jax
import jax.numpy as jnp
from jax import lax
from jax.experimental import pallas as pl
from jax.experimental.pallas import tpu as pltpu, tpu_sc as plsc

B, N_C, N_T, D_S, D_F, K, D_OBS, H = 16, 1024, 512, 2, 1, 16, 4, 64
NE = B * N_T * K
NW = 32
EPW = NE // NW
NPW = B * N_T // NW
GE = 128
GN = GE // K
NG = EPW // GE


def _mlp3_ln(x, w1, b1, w2, b2, w3, b3, ln_s, ln_b):
    x = jax.nn.gelu(jnp.dot(x, w1, preferred_element_type=jnp.float32) + b1)
    x = jax.nn.gelu(jnp.dot(x, w2, preferred_element_type=jnp.float32) + b2)
    x = jnp.dot(x, w3, preferred_element_type=jnp.float32) + b3
    mu = jnp.mean(x, axis=-1, keepdims=True)
    var = jnp.mean((x - mu) ** 2, axis=-1, keepdims=True)
    return (x - mu) / jnp.sqrt(var + 1e-6) * ln_s + ln_b


def _front_batch(ctxf_ref, tstf_ref, sctx_ref, sctxt_ref, stestt_ref, w1_ref,
                 b1_ref, w2_ref, b2_ref, w3_ref, b3_ref, lns_ref, lnb_ref,
                 gatw_ref, asrc_ref, adst_ref, misc_ref,
                 ntst_ref, hctx_ref, idx_ref, attn_ref):
    ctxf = ctxf_ref[0]
    tstf = tstf_ref[0]
    xkc = sctx_ref[0, :, 0:1]
    ykc = sctx_ref[0, :, 1:2]
    xk = sctxt_ref[0, 0:1, :]
    yk = sctxt_ref[0, 1:2, :]
    xq = stestt_ref[0, 0:1, :]
    yq = stestt_ref[0, 1:2, :]

    w1, b1 = w1_ref[...], b1_ref[...]
    w2, b2 = w2_ref[...], b2_ref[...]
    w3, b3 = w3_ref[...], b3_ref[...]
    ln_s, ln_b = lns_ref[...], lnb_ref[...]

    n_ctx = _mlp3_ln(ctxf, w1, b1, w2, b2, w3, b3, ln_s, ln_b)
    n_tst = _mlp3_ln(tstf, w1, b1, w2, b2, w3, b3, ln_s, ln_b)
    ntst_ref[0] = n_tst

    gat_w = gatw_ref[...]
    h_ctx = jnp.dot(n_ctx, gat_w, preferred_element_type=jnp.float32)
    h_tst = jnp.dot(n_tst, gat_w, preferred_element_type=jnp.float32)
    hctx_ref[0] = h_ctx
    asrc_p = asrc_ref[...]
    adst_p = adst_ref[...]
    hsrc_r = lax.dot_general(asrc_p, h_ctx, (((1,), (1,)), ((), ())),
                             preferred_element_type=jnp.float32)
    hdst_r = lax.dot_general(adst_p, h_tst, (((1,), (1,)), ((), ())),
                             preferred_element_type=jnp.float32)
    hdst = hdst_r[0:1, :]

    gtab = jnp.concatenate(
        [xk, yk, hsrc_r[0:1, :], jnp.zeros((5, N_C), jnp.float32)], axis=0)

    d2 = (xkc - xq) ** 2 + (ykc - yq) ** 2
    iota = lax.broadcasted_iota(
        jnp.int32, (N_C, N_T), 0).astype(jnp.float32)
    bigi = jnp.float32(2e9)
    bigv = jnp.float32(1e30)

    idxs = []
    gath = []
    for _ in range(K):
        m = jnp.min(d2, axis=0, keepdims=True)
        cand = jnp.where(d2 == m, iota, bigi)
        idx = jnp.min(cand, axis=0, keepdims=True)
        e = iota == idx
        ef = jnp.where(e, 1.0, 0.0)
        idxs.append(idx)
        gath.append(jnp.dot(gtab, ef,
                            preferred_element_type=jnp.float32))
        d2 = jnp.where(e, bigv, d2)

    nbx = jnp.concatenate([g[0:1, :] for g in gath], axis=0)
    nby = jnp.concatenate([g[1:2, :] for g in gath], axis=0)
    hs = jnp.concatenate([g[2:3, :] for g in gath], axis=0)
    idxk = jnp.concatenate(idxs, axis=0)

    ew0 = misc_ref[0, 0]
    ew1 = misc_ref[0, 1]
    eb = misc_ref[0, 2]
    ebias = (xq - nbx) * ew0 + (yq - nby) * ew1
    z = hs + hdst
    logit = jnp.where(z >= 0, z, 0.2 * z) + ebias + eb
    mrow = jnp.max(logit, axis=0, keepdims=True)
    p = jnp.exp(logit - mrow)
    attn_ref[0] = p / (jnp.sum(p, axis=0, keepdims=True) + 1e-9)

    boff = (pl.program_id(0) * N_C).astype(jnp.float32)
    idx_ref[0] = (idxk + boff).astype(jnp.int32)


def _head_batch(ntst_ref, agg_ref, hw1_ref, hb1_ref, hw2_ref, hb2_ref,
                hw3_ref, hb3_ref, out_ref):
    new_t = ntst_ref[0] + agg_ref[0]
    x = jax.nn.gelu(jnp.dot(new_t, hw1_ref[...],
                            preferred_element_type=jnp.float32) + hb1_ref[...])
    x = jax.nn.gelu(jnp.dot(x, hw2_ref[...],
                            preferred_element_type=jnp.float32) + hb2_ref[...])
    f_dist = jnp.dot(x, hw3_ref[...],
                     preferred_element_type=jnp.float32) + hb3_ref[...]
    col = lax.broadcasted_iota(jnp.int32, (N_T, 8), 1)
    soft = jnp.logaddexp(f_dist, 0.0) + 1e-3
    out_ref[0] = jnp.where(col == 0, f_dist, soft)


def _sc_agg(h_hbm, idx_hbm, aw_hbm, out_hbm, idxv, awv, rowsv, outv, sem):
    wid = lax.axis_index("s") * 2 + lax.axis_index("c")

    @pl.loop(0, NG)
    def group(g):
        base_e = wid * EPW + g * GE
        base_n = wid * NPW + g * GN
        pltpu.sync_copy(idx_hbm.at[pl.ds(base_e, GE)], idxv)
        pltpu.sync_copy(aw_hbm.at[pl.ds(base_e, GE)], awv)
        pltpu.async_copy(h_hbm.at[idxv], rowsv, sem).wait()
        for i in range(GN):
            acc = [jnp.zeros((16,), jnp.float32) for _ in range(4)]
            for j in range(K):
                ej = i * K + j
                w = awv[ej, pl.ds(0, 16)]
                for q in range(4):
                    acc[q] = acc[q] + w * rowsv[ej, pl.ds(16 * q, 16)]
            for q in range(4):
                outv[i, pl.ds(16 * q, 16)] = acc[q]
        pltpu.sync_copy(outv, out_hbm.at[pl.ds(base_n, GN)])


def kernel(s_ctx, f_ctx, s_test, emb_obs, W1, b1, W2, b2, W3, b3, ln_s, ln_b,
           gat_W, a_src, a_dst, e_w, e_b, hW1, hb1, hW2, hb2, hW3, hb3):
    f32 = jnp.float32
    obs_c = jnp.broadcast_to(emb_obs[1], (B, N_C, D_OBS))
    obs_t = jnp.broadcast_to(emb_obs[0], (B, N_T, D_OBS))
    ctxf = jnp.concatenate(
        [obs_c, s_ctx, f_ctx, jnp.zeros((B, N_C, 1), f32)], axis=-1)
    tstf = jnp.concatenate(
        [obs_t, s_test, jnp.zeros((B, N_T, 2), f32)], axis=-1)
    sctxt = jnp.transpose(s_ctx, (0, 2, 1))
    stestt = jnp.transpose(s_test, (0, 2, 1))

    w1p = jnp.concatenate([W1, jnp.zeros((1, 256), f32)], axis=0)
    asrc_p = jnp.concatenate([a_src[None, :], jnp.zeros((7, H), f32)], axis=0)
    adst_p = jnp.concatenate([a_dst[None, :], jnp.zeros((7, H), f32)], axis=0)
    misc = jnp.stack([e_w[0], e_w[1], e_b, jnp.zeros((), f32)])[None, :]
    hw3p = jnp.concatenate([hW3, jnp.zeros((64, 6), f32)], axis=1)
    hb3p = jnp.concatenate([hb3, jnp.zeros((6,), f32)])[None, :]

    full = lambda shape: pl.BlockSpec(shape, lambda b: (0,) * len(shape))
    per_b3 = lambda s1, s2: pl.BlockSpec((1, s1, s2), lambda b: (b, 0, 0))

    n_tst, h_ctx, idxg, attn = pl.pallas_call(
        _front_batch,
        grid=(B,),
        in_specs=[
            per_b3(N_C, 8), per_b3(N_T, 8), per_b3(N_C, 2), per_b3(2, N_C),
            per_b3(2, N_T),
            full((8, 256)), full((1, 256)), full((256, 128)), full((1, 128)),
            full((128, H)), full((1, H)), full((1, H)), full((1, H)),
            full((H, H)), full((8, H)), full((8, H)), full((1, 4)),
        ],
        out_specs=[per_b3(N_T, H), per_b3(N_C, H), per_b3(K, N_T),
                   per_b3(K, N_T)],
        out_shape=[jax.ShapeDtypeStruct((B, N_T, H), f32),
                   jax.ShapeDtypeStruct((B, N_C, H), f32),
                   jax.ShapeDtypeStruct((B, K, N_T), jnp.int32),
                   jax.ShapeDtypeStruct((B, K, N_T), f32)],
    )(ctxf, tstf, s_ctx, sctxt, stestt, w1p, b1[None, :], W2, b2[None, :], W3,
      b3[None, :], ln_s[None, :], ln_b[None, :], gat_W, asrc_p, adst_p, misc)

    h2 = jnp.pad(h_ctx.reshape(B * N_C, H), ((0, 0), (0, 128 - H)))
    ie = jnp.transpose(idxg, (0, 2, 1)).reshape(-1)
    ae = jnp.transpose(attn, (0, 2, 1)).reshape(-1)
    aex = jnp.broadcast_to(ae[:, None], (NE, 16))

    mesh = plsc.VectorSubcoreMesh(core_axis_name="c", subcore_axis_name="s")
    agg_fn = functools.partial(
        pl.kernel, mesh=mesh,
        out_type=jax.ShapeDtypeStruct((B * N_T, H), f32),
        scratch_types=[
            pltpu.VMEM((GE,), jnp.int32),
            pltpu.VMEM((GE, 16), f32),
            pltpu.VMEM((GE, 128), f32),
            pltpu.VMEM((GN, H), f32),
            pltpu.SemaphoreType.DMA,
        ],
    )(_sc_agg)
    agg = agg_fn(h2, ie, aex).reshape(B, N_T, H)

    out = pl.pallas_call(
        _head_batch,
        grid=(B,),
        in_specs=[
            per_b3(N_T, H), per_b3(N_T, H),
            full((H, 256)), full((1, 256)), full((256, 64)), full((1, 64)),
            full((64, 8)), full((1, 8)),
        ],
        out_specs=per_b3(N_T, 8),
        out_shape=jax.ShapeDtypeStruct((B, N_T, 8), f32),
    )(n_tst, agg, hW1, hb1[None, :], hW2, hb2[None, :], hw3p, hb3p)
    return out[:, :, :2]


# speedup vs baseline: 1.5912x; 1.5912x over previous
import jax
import jax.numpy as jnp
from jax.experimental import pallas as pl

B, N_C, N_T, D_S, D_F, K, D_OBS, H = 16, 1024, 512, 2, 1, 16, 4, 64


def _mlp3_ln(x, w1, b1, w2, b2, w3, b3, ln_s, ln_b):
    x = jax.nn.gelu(jnp.dot(x, w1, preferred_element_type=jnp.float32) + b1)
    x = jax.nn.gelu(jnp.dot(x, w2, preferred_element_type=jnp.float32) + b2)
    x = jnp.dot(x, w3, preferred_element_type=jnp.float32) + b3
    mu = jnp.mean(x, axis=-1, keepdims=True)
    var = jnp.mean((x - mu) ** 2, axis=-1, keepdims=True)
    return (x - mu) / jnp.sqrt(var + 1e-6) * ln_s + ln_b


def _sgnp_batch(ctxf_ref, tstf_ref, sctx_ref, sctxt_ref, stestt_ref, w1_ref,
                b1_ref, w2_ref, b2_ref, w3_ref, b3_ref, lns_ref, lnb_ref,
                gatw_ref, asrc_ref, adst_ref, misc_ref, hw1_ref, hb1_ref,
                hw2_ref, hb2_ref, hw3_ref, hb3_ref, out_ref):
    ctxf = ctxf_ref[0]
    tstf = tstf_ref[0]
    xk = sctxt_ref[0, 0:1, :]
    yk = sctxt_ref[0, 1:2, :]
    xkc = sctx_ref[0, :, 0:1]
    ykc = sctx_ref[0, :, 1:2]
    xq = stestt_ref[0, 0:1, :]
    yq = stestt_ref[0, 1:2, :]

    w1, b1 = w1_ref[...], b1_ref[...]
    w2, b2 = w2_ref[...], b2_ref[...]
    w3, b3 = w3_ref[...], b3_ref[...]
    ln_s, ln_b = lns_ref[...], lnb_ref[...]

    n_ctx = _mlp3_ln(ctxf, w1, b1, w2, b2, w3, b3, ln_s, ln_b)
    n_tst = _mlp3_ln(tstf, w1, b1, w2, b2, w3, b3, ln_s, ln_b)

    gat_w = gatw_ref[...]
    h_ctx = jnp.dot(n_ctx, gat_w, preferred_element_type=jnp.float32)
    h_tst = jnp.dot(n_tst, gat_w, preferred_element_type=jnp.float32)
    asrc_p = asrc_ref[...]
    adst_p = adst_ref[...]
    hsrc_r = jax.lax.dot_general(asrc_p, h_ctx, (((1,), (1,)), ((), ())),
                                 preferred_element_type=jnp.float32)
    hdst_r = jax.lax.dot_general(adst_p, h_tst, (((1,), (1,)), ((), ())),
                                 preferred_element_type=jnp.float32)
    hdst = hdst_r[0:1, :]

    gtab = jnp.concatenate(
        [xk, yk, hsrc_r[0:1, :], jnp.zeros((5, N_C), jnp.float32)], axis=0)

    d2 = (xkc - xq) ** 2 + (ykc - yq) ** 2
    iota = jax.lax.broadcasted_iota(
        jnp.int32, (N_C, N_T), 0).astype(jnp.float32)
    bigi = jnp.float32(2e9)
    bigv = jnp.float32(1e30)

    idxs = []
    gath = []
    for _ in range(K):
        m = jnp.min(d2, axis=0, keepdims=True)
        cand = jnp.where(d2 == m, iota, bigi)
        idx = jnp.min(cand, axis=0, keepdims=True)
        e = iota == idx
        ef = jnp.where(e, 1.0, 0.0)
        idxs.append(idx)
        gath.append(jnp.dot(gtab, ef,
                            preferred_element_type=jnp.float32))
        d2 = jnp.where(e, bigv, d2)

    nbx = jnp.concatenate([g[0:1, :] for g in gath], axis=0)
    nby = jnp.concatenate([g[1:2, :] for g in gath], axis=0)
    hs = jnp.concatenate([g[2:3, :] for g in gath], axis=0)

    ew0 = misc_ref[0, 0]
    ew1 = misc_ref[0, 1]
    eb = misc_ref[0, 2]
    ebias = (xq - nbx) * ew0 + (yq - nby) * ew1
    z = hs + hdst
    logit = jnp.where(z >= 0, z, 0.2 * z) + ebias + eb
    mrow = jnp.max(logit, axis=0, keepdims=True)
    p = jnp.exp(logit - mrow)
    attn = p / (jnp.sum(p, axis=0, keepdims=True) + 1e-9)

    wsel = jnp.zeros((N_C, N_T), jnp.float32)
    for k in range(K):
        wsel = jnp.where(iota == idxs[k], attn[k:k + 1, :], wsel)
    agg = jax.lax.dot_general(wsel, h_ctx, (((0,), (0,)), ((), ())),
                              preferred_element_type=jnp.float32)

    new_t = n_tst + agg
    hkw1, hkb1 = hw1_ref[...], hb1_ref[...]
    hkw2, hkb2 = hw2_ref[...], hb2_ref[...]
    hkw3, hkb3 = hw3_ref[...], hb3_ref[...]
    x = jax.nn.gelu(jnp.dot(new_t, hkw1, preferred_element_type=jnp.float32) + hkb1)
    x = jax.nn.gelu(jnp.dot(x, hkw2, preferred_element_type=jnp.float32) + hkb2)
    f_dist = jnp.dot(x, hkw3, preferred_element_type=jnp.float32) + hkb3
    col = jax.lax.broadcasted_iota(jnp.int32, (N_T, 8), 1)
    soft = jnp.logaddexp(f_dist, 0.0) + 1e-3
    out_ref[0] = jnp.where(col == 0, f_dist, soft)


def kernel(s_ctx, f_ctx, s_test, emb_obs, W1, b1, W2, b2, W3, b3, ln_s, ln_b,
           gat_W, a_src, a_dst, e_w, e_b, hW1, hb1, hW2, hb2, hW3, hb3):
    f32 = jnp.float32
    obs_c = jnp.broadcast_to(emb_obs[1], (B, N_C, D_OBS))
    obs_t = jnp.broadcast_to(emb_obs[0], (B, N_T, D_OBS))
    ctxf = jnp.concatenate(
        [obs_c, s_ctx, f_ctx, jnp.zeros((B, N_C, 1), f32)], axis=-1)
    tstf = jnp.concatenate(
        [obs_t, s_test, jnp.zeros((B, N_T, 2), f32)], axis=-1)
    sctxt = jnp.transpose(s_ctx, (0, 2, 1))
    stestt = jnp.transpose(s_test, (0, 2, 1))

    w1p = jnp.concatenate([W1, jnp.zeros((1, 256), f32)], axis=0)
    asrc_p = jnp.concatenate([a_src[None, :], jnp.zeros((7, H), f32)], axis=0)
    adst_p = jnp.concatenate([a_dst[None, :], jnp.zeros((7, H), f32)], axis=0)
    misc = jnp.stack([e_w[0], e_w[1], e_b, jnp.zeros((), f32)])[None, :]
    hw3p = jnp.concatenate([hW3, jnp.zeros((64, 6), f32)], axis=1)
    hb3p = jnp.concatenate([hb3, jnp.zeros((6,), f32)])[None, :]

    full = lambda shape: pl.BlockSpec(shape, lambda b: (0,) * len(shape))
    per_b3 = lambda s1, s2: pl.BlockSpec((1, s1, s2), lambda b: (b, 0, 0))

    out = pl.pallas_call(
        _sgnp_batch,
        grid=(B,),
        in_specs=[
            per_b3(N_C, 8), per_b3(N_T, 8), per_b3(N_C, 2), per_b3(2, N_C),
            per_b3(2, N_T),
            full((8, 256)), full((1, 256)), full((256, 128)), full((1, 128)),
            full((128, H)), full((1, H)), full((1, H)), full((1, H)),
            full((H, H)), full((8, H)), full((8, H)), full((1, 4)),
            full((H, 256)), full((1, 256)), full((256, 64)), full((1, 64)),
            full((64, 8)), full((1, 8)),
        ],
        out_specs=per_b3(N_T, 8),
        out_shape=jax.ShapeDtypeStruct((B, N_T, 8), f32),
    )(ctxf, tstf, s_ctx, sctxt, stestt, w1p, b1[None, :], W2, b2[None, :], W3,
      b3[None, :], ln_s[None, :], ln_b[None, :], gat_W, asrc_p, adst_p, misc,
      hW1, hb1[None, :], hW2, hb2[None, :], hw3p, hb3p)
    return out[:, :, :2]
